# Initial kernel scaffold; baseline (speedup 1.0000x reference)
import jax
import jax.numpy as jnp
from jax.experimental import pallas as pl


def kernel(obs, ego_init, other_init, edge_index):
    raise NotImplementedError("write your pallas kernel here")


# speedup vs baseline: 1.6181x; 1.6181x over previous
import functools

import numpy as np

import jax
import jax.numpy as jnp
from jax import lax
from jax.experimental import pallas as pl
from jax.experimental.pallas import tpu as pltpu
from jax.experimental.pallas import tpu_sc as plsc

_INPUT_DIM = 4
_L = 16


@functools.lru_cache(maxsize=None)
def _build_sc_call(batch, node_num, input_dim, init_dim, num_edges):
    nc, ns = 2, 16
    nw = nc * ns
    assert batch % (8 * nw) == 0
    nb = batch // nw
    obs_row = node_num * input_dim
    out_d = input_dim + init_dim
    row_words = node_num * out_d
    obs_words = nb * obs_row
    out_words = nb * row_words
    ei_words = nb * num_edges
    obs_off = 8
    import math
    lcm_words = math.lcm(_L, row_words)
    nres = lcm_words // _L
    bat_per_macro = lcm_words // row_words
    n_macro = out_words // lcm_words
    obs_step = obs_row * bat_per_macro
    assert _L % num_edges == 0
    ei_bpv = _L // num_edges
    ei_step = node_num * ei_bpv
    n_ei = ei_words // _L

    lane_np = np.arange(_L)
    bases_np, steps_np = [], []
    for r in range(nres):
        j0 = lane_np + _L * r
        bb0 = j0 // row_words
        p = j0 % row_words
        n = p // out_d
        c = p % out_d
        is_obs = c < input_dim
        base_v = np.where(is_obs,
                          obs_off + obs_row * bb0 + input_dim * n + c,
                          np.where(n > 0, init_dim, 0) + (c - input_dim))
        bases_np.append(base_v.astype(np.int32))
        steps_np.append(np.where(is_obs, obs_step, 0).astype(np.int32))
    ei_tri_np = ((lane_np // num_edges) * node_num).astype(np.int32)
    ei_idx0_np = (lane_np % num_edges).astype(np.int32)
    ei_idx1_np = (num_edges + lane_np % num_edges).astype(np.int32)
    consts_np = np.concatenate(
        bases_np + steps_np + [ei_tri_np, ei_idx0_np, ei_idx1_np])
    n_const = consts_np.shape[0]

    mesh = plsc.VectorSubcoreMesh(core_axis_name="c", subcore_axis_name="s")

    @functools.partial(
        pl.kernel,
        out_type=[
            jax.ShapeDtypeStruct((batch * row_words,), jnp.float32),
            jax.ShapeDtypeStruct((2 * batch * num_edges,), jnp.int32),
        ],
        mesh=mesh,
        compiler_params=pltpu.CompilerParams(needs_layout_passes=False),
        scratch_types=[
            pltpu.VMEM((obs_off + obs_words,), jnp.float32),
            pltpu.VMEM((out_words,), jnp.float32),
            pltpu.VMEM((2 * num_edges,), jnp.int32),
            pltpu.VMEM((2 * ei_words,), jnp.int32),
            pltpu.VMEM((n_const,), jnp.int32),
        ],
    )
    def sc_call(obs_hbm, init_hbm, ei_hbm, consts_hbm, x_hbm, eiout_hbm,
                stage, outbuf, eist, eibuf, constv):
        wid = lax.axis_index("s") * nc + lax.axis_index("c")
        base = wid * nb
        pltpu.sync_copy(init_hbm, stage.at[pl.ds(0, 2 * init_dim)])
        pltpu.sync_copy(ei_hbm, eist)
        pltpu.sync_copy(consts_hbm, constv)
        pltpu.sync_copy(obs_hbm.at[pl.ds(base * obs_row, obs_words)],
                        stage.at[pl.ds(obs_off, obs_words)])

        bases = [constv[pl.ds(_L * r, _L)] for r in range(nres)]
        steps = [constv[pl.ds(_L * (nres + r), _L)] for r in range(nres)]

        def xbody(mi, carry):
            out0 = mi * lcm_words
            for r in range(nres):
                idx = bases[r] + mi * steps[r]
                outbuf[pl.ds(out0 + _L * r, _L)] = plsc.load_gather(stage, [idx])
            return carry
        lax.fori_loop(0, n_macro, xbody, 0, unroll=False)

        tri = constv[pl.ds(_L * 2 * nres, _L)] + node_num * base
        b0 = plsc.load_gather(eist, [constv[pl.ds(_L * (2 * nres + 1), _L)]]) + tri
        b1 = plsc.load_gather(eist, [constv[pl.ds(_L * (2 * nres + 2), _L)]]) + tri

        def eibody(k, carry):
            d = k * ei_step
            eibuf[pl.ds(_L * k, _L)] = b0 + d
            eibuf[pl.ds(ei_words + _L * k, _L)] = b1 + d
            return carry
        lax.fori_loop(0, n_ei, eibody, 0, unroll=False)

        pltpu.sync_copy(outbuf, x_hbm.at[pl.ds(base * row_words, out_words)])
        pltpu.sync_copy(eibuf.at[pl.ds(0, ei_words)],
                        eiout_hbm.at[pl.ds(base * num_edges, ei_words)])
        pltpu.sync_copy(eibuf.at[pl.ds(ei_words, ei_words)],
                        eiout_hbm.at[pl.ds((batch + base) * num_edges, ei_words)])

    return sc_call, consts_np


def kernel(obs, ego_init, other_init, edge_index):
    batch, obs_dim = obs.shape
    node_num = obs_dim // _INPUT_DIM
    init_dim = ego_init.shape[0]
    num_edges = edge_index.shape[1]
    sc_call, consts_np = _build_sc_call(batch, node_num, _INPUT_DIM,
                                        init_dim, num_edges)
    x_flat, ei_flat = sc_call(obs.reshape(-1),
                              jnp.concatenate([ego_init, other_init]),
                              edge_index.reshape(-1),
                              jnp.asarray(consts_np))
    return (x_flat.reshape(batch * node_num, _INPUT_DIM + init_dim),
            ei_flat.reshape(2, batch * num_edges))


# speedup vs baseline: 7.8785x; 4.8691x over previous
import functools

import numpy as np

import jax
import jax.numpy as jnp
from jax import lax
from jax.experimental import pallas as pl
from jax.experimental.pallas import tpu as pltpu
from jax.experimental.pallas import tpu_sc as plsc

_INPUT_DIM = 4
_L = 16


@functools.lru_cache(maxsize=None)
def _build_sc_call(batch, node_num, input_dim, init_dim, num_edges):
    nc, ns = 2, 16
    nw = nc * ns
    nb = batch // nw
    obs_row = node_num * input_dim
    out_d = input_dim + init_dim
    ncols = batch * node_num
    assert ncols % (128 * nw) == 0 and (ncols // nw) % (3 * 128) == 0
    wtiles = ncols // 128 // nw
    band = (batch // 128) * 1024
    obs_chunk = (nb // 128) * 1024
    ei_words = nb * num_edges
    obs_off = 8
    n_macro = wtiles // node_num
    ei_step = node_num * (_L // num_edges)
    n_ei = ei_words // _L

    lane = np.arange(_L)
    tabs = []
    for d in range(input_dim):
        for tr in range(node_num):
            for s in range(128 // _L):
                cc = _L * s + lane
                bl = (128 * tr + cc) // node_num
                n = (2 * tr + cc) % node_num
                q = input_dim * n + d
                tabs.append((obs_off + (q // 8) * obs_chunk
                             + (bl // 128) * 1024 + (q % 8) * 128
                             + bl % 128).astype(np.int32))
    for d in range(out_d - input_dim):
        for rr in range(node_num):
            n = (rr + lane) % node_num
            tabs.append(np.where(n == 0, d, init_dim + d).astype(np.int32))
    tabs.append(((lane // num_edges) * node_num).astype(np.int32))
    tabs.append((lane % num_edges).astype(np.int32))
    tabs.append((num_edges + lane % num_edges).astype(np.int32))
    consts_np = np.concatenate(tabs)
    n_const = consts_np.shape[0]
    n_xvec = input_dim * node_num * (128 // _L)

    mesh = plsc.VectorSubcoreMesh(core_axis_name="c", subcore_axis_name="s")

    @functools.partial(
        pl.kernel,
        out_type=[
            jax.ShapeDtypeStruct((8 * ncols,), jnp.float32),
            jax.ShapeDtypeStruct((2 * batch * num_edges,), jnp.int32),
        ],
        mesh=mesh,
        compiler_params=pltpu.CompilerParams(needs_layout_passes=False),
        scratch_types=[
            pltpu.VMEM((obs_off + 2 * obs_chunk,), jnp.float32),
            pltpu.VMEM((wtiles * 1024,), jnp.float32),
            pltpu.VMEM((2 * num_edges,), jnp.int32),
            pltpu.VMEM((2 * ei_words,), jnp.int32),
            pltpu.VMEM((n_const,), jnp.int32),
        ],
    )
    def sc_call(obsp_hbm, init_hbm, ei_hbm, consts_hbm, x_hbm, eiout_hbm,
                stage, outbuf, eist, eibuf, constv):
        wid = lax.axis_index("s") * nc + lax.axis_index("c")
        base = wid * nb
        pltpu.sync_copy(init_hbm, stage.at[pl.ds(0, 2 * init_dim)])
        pltpu.sync_copy(ei_hbm, eist)
        pltpu.sync_copy(consts_hbm, constv)
        for i in range(2):
            pltpu.sync_copy(
                obsp_hbm.at[pl.ds(i * band + wid * obs_chunk, obs_chunk)],
                stage.at[pl.ds(obs_off + i * obs_chunk, obs_chunk)])

        for d in range(input_dim):
            kb = [constv[pl.ds(_L * (d * 24 + j), _L)] for j in range(24)]

            def xbody(mi, carry, _d=d, _kb=kb):
                mo = mi * (node_num * 1024) + _d * 128
                ms = mi * 1024
                for tr in range(node_num):
                    for s in range(128 // _L):
                        idx = _kb[tr * 8 + s] + ms
                        outbuf[pl.ds(mo + tr * 1024 + _L * s, _L)] = \
                            plsc.load_gather(stage, [idx])
                return carry
            lax.fori_loop(0, n_macro, xbody, 0, unroll=False)

        for d in range(input_dim, out_d):
            vals = [plsc.load_gather(
                        stage,
                        [constv[pl.ds(_L * (n_xvec + (d - input_dim)
                                            * node_num + rr), _L)]])
                    for rr in range(node_num)]

            def ibody(mi, carry, _d=d, _vals=vals):
                mo = mi * (node_num * 1024) + _d * 128
                for tr in range(node_num):
                    for s in range(128 // _L):
                        rr = (2 * tr + _L * s) % node_num
                        outbuf[pl.ds(mo + tr * 1024 + _L * s, _L)] = _vals[rr]
                return carry
            lax.fori_loop(0, n_macro, ibody, 0, unroll=False)

        pltpu.sync_copy(outbuf,
                        x_hbm.at[pl.ds(wid * (wtiles * 1024), wtiles * 1024)])

        ce = n_xvec + 2 * node_num
        tri = constv[pl.ds(_L * ce, _L)] + node_num * base
        b0 = plsc.load_gather(eist, [constv[pl.ds(_L * (ce + 1), _L)]]) + tri
        b1 = plsc.load_gather(eist, [constv[pl.ds(_L * (ce + 2), _L)]]) + tri

        def eibody(k, carry):
            dd = k * ei_step
            eibuf[pl.ds(_L * k, _L)] = b0 + dd
            eibuf[pl.ds(ei_words + _L * k, _L)] = b1 + dd
            return carry
        lax.fori_loop(0, n_ei, eibody, 0, unroll=False)

        pltpu.sync_copy(eibuf.at[pl.ds(0, ei_words)],
                        eiout_hbm.at[pl.ds(base * num_edges, ei_words)])
        pltpu.sync_copy(eibuf.at[pl.ds(ei_words, ei_words)],
                        eiout_hbm.at[pl.ds((batch + base) * num_edges, ei_words)])

    return sc_call, consts_np


def kernel(obs, ego_init, other_init, edge_index):
    batch, obs_dim = obs.shape
    node_num = obs_dim // _INPUT_DIM
    init_dim = ego_init.shape[0]
    num_edges = edge_index.shape[1]
    sc_call, consts_np = _build_sc_call(batch, node_num, _INPUT_DIM,
                                        init_dim, num_edges)
    nrow = node_num * _INPUT_DIM
    pad_r = -nrow % 8
    ncols = batch * node_num
    obsp = jnp.concatenate(
        [obs.T, jnp.zeros((pad_r, batch), obs.dtype)], axis=0)
    obsp = obsp.reshape((nrow + pad_r) // 8, 8, batch // 128, 128) \
               .transpose(0, 2, 1, 3).reshape(-1)
    xphys, ei_flat = sc_call(obsp,
                             jnp.concatenate([ego_init, other_init]),
                             edge_index.reshape(-1),
                             jnp.asarray(consts_np))
    x = xphys.reshape(ncols // 128, 8, 128).transpose(1, 0, 2) \
             .reshape(8, ncols)[:_INPUT_DIM + init_dim].T
    return x, ei_flat.reshape(2, batch * num_edges)


# speedup vs baseline: 8.4075x; 1.0671x over previous
import functools

import numpy as np

import jax
import jax.numpy as jnp
from jax import lax
from jax.experimental import pallas as pl
from jax.experimental.pallas import tpu as pltpu
from jax.experimental.pallas import tpu_sc as plsc

_INPUT_DIM = 4
_L = 16


@functools.lru_cache(maxsize=None)
def _build_sc_call(batch, node_num, input_dim, init_dim, num_edges):
    nc, ns = 2, 16
    nw = nc * ns
    nb = batch // nw
    obs_row = node_num * input_dim
    out_d = input_dim + init_dim
    ncols = batch * node_num
    assert ncols % (128 * nw) == 0 and (ncols // nw) % (3 * 128) == 0
    wtiles = ncols // 128 // nw
    band = (batch // 128) * 1024
    obs_chunk = (nb // 128) * 1024
    ei_words = nb * num_edges
    obs_off = 8
    n_macro = wtiles // node_num
    ei_step = node_num * (_L // num_edges)
    n_ei = ei_words // _L

    lane = np.arange(_L)
    tabs = []
    for d in range(input_dim):
        for tr in range(node_num):
            for s in range(128 // _L):
                cc = _L * s + lane
                bl = (128 * tr + cc) // node_num
                n = (2 * tr + cc) % node_num
                q = input_dim * n + d
                tabs.append((obs_off + (q // 8) * obs_chunk
                             + (bl // 128) * 1024 + (q % 8) * 128
                             + bl % 128).astype(np.int32))
    for d in range(out_d - input_dim):
        for rr in range(node_num):
            n = (rr + lane) % node_num
            tabs.append(np.where(n == 0, d, init_dim + d).astype(np.int32))
    tabs.append(((lane // num_edges) * node_num).astype(np.int32))
    tabs.append((lane % num_edges).astype(np.int32))
    tabs.append((num_edges + lane % num_edges).astype(np.int32))
    consts_np = np.concatenate(tabs)
    n_const = consts_np.shape[0]
    n_xvec = input_dim * node_num * (128 // _L)

    mesh = plsc.VectorSubcoreMesh(core_axis_name="c", subcore_axis_name="s")

    @functools.partial(
        pl.kernel,
        out_type=[
            jax.ShapeDtypeStruct((8 * ncols,), jnp.float32),
            jax.ShapeDtypeStruct((2 * batch * num_edges,), jnp.int32),
        ],
        mesh=mesh,
        compiler_params=pltpu.CompilerParams(needs_layout_passes=False),
        scratch_types=[
            pltpu.VMEM((obs_off + 2 * obs_chunk,), jnp.float32),
            pltpu.VMEM((wtiles * 1024,), jnp.float32),
            pltpu.VMEM((2 * num_edges,), jnp.int32),
            pltpu.VMEM((2 * ei_words,), jnp.int32),
            pltpu.VMEM((n_const,), jnp.int32),
            pltpu.SemaphoreType.DMA,
            pltpu.SemaphoreType.DMA,
            pltpu.SemaphoreType.DMA,
        ],
    )
    def sc_call(obsp_hbm, init_hbm, ei_hbm, consts_hbm, x_hbm, eiout_hbm,
                stage, outbuf, eist, eibuf, constv, sem_a, sem_b, sem_o):
        wid = lax.axis_index("s") * nc + lax.axis_index("c")
        base = wid * nb
        h_small = [
            pltpu.async_copy(init_hbm, stage.at[pl.ds(0, 2 * init_dim)],
                             sem_a),
            pltpu.async_copy(ei_hbm, eist, sem_a),
            pltpu.async_copy(consts_hbm, constv, sem_a),
        ]
        h_obs = [
            pltpu.async_copy(
                obsp_hbm.at[pl.ds(i * band + wid * obs_chunk, obs_chunk)],
                stage.at[pl.ds(obs_off + i * obs_chunk, obs_chunk)], sem_b)
            for i in range(2)]
        for h in h_small:
            h.wait()

        ce = n_xvec + 2 * node_num
        tri = constv[pl.ds(_L * ce, _L)] + node_num * base
        b0 = plsc.load_gather(eist, [constv[pl.ds(_L * (ce + 1), _L)]]) + tri
        b1 = plsc.load_gather(eist, [constv[pl.ds(_L * (ce + 2), _L)]]) + tri

        def eibody(k, carry):
            dd = k * ei_step
            eibuf[pl.ds(_L * k, _L)] = b0 + dd
            eibuf[pl.ds(ei_words + _L * k, _L)] = b1 + dd
            return carry
        lax.fori_loop(0, n_ei, eibody, 0, unroll=8)

        h_out = [
            pltpu.async_copy(eibuf.at[pl.ds(0, ei_words)],
                             eiout_hbm.at[pl.ds(base * num_edges, ei_words)],
                             sem_o),
            pltpu.async_copy(
                eibuf.at[pl.ds(ei_words, ei_words)],
                eiout_hbm.at[pl.ds((batch + base) * num_edges, ei_words)],
                sem_o),
        ]
        for h in h_obs:
            h.wait()

        xbase = wid * (wtiles * 1024)
        for lo, hi in ((0, n_macro // 2), (n_macro // 2, n_macro)):
            for d in range(input_dim):
                kb = [constv[pl.ds(_L * (d * 24 + j), _L)] for j in range(24)]

                def xbody(mi, carry, _d=d, _kb=kb):
                    mo = mi * (node_num * 1024) + _d * 128
                    ms = mi * 1024
                    for tr in range(node_num):
                        for s in range(128 // _L):
                            idx = _kb[tr * 8 + s] + ms
                            outbuf[pl.ds(mo + tr * 1024 + _L * s, _L)] = \
                                plsc.load_gather(stage, [idx])
                    return carry
                lax.fori_loop(lo, hi, xbody, 0, unroll=False)

            for d in range(input_dim, out_d):
                vals = [plsc.load_gather(
                            stage,
                            [constv[pl.ds(_L * (n_xvec + (d - input_dim)
                                                * node_num + rr), _L)]])
                        for rr in range(node_num)]

                def ibody(mi, carry, _d=d, _vals=vals):
                    mo = mi * (node_num * 1024) + _d * 128
                    for tr in range(node_num):
                        for s in range(128 // _L):
                            rr = (2 * tr + _L * s) % node_num
                            outbuf[pl.ds(mo + tr * 1024 + _L * s, _L)] = \
                                _vals[rr]
                    return carry
                lax.fori_loop(lo, hi, ibody, 0, unroll=False)

            for t in range(lo * node_num, hi * node_num):
                h_out.append(pltpu.async_copy(
                    outbuf.at[pl.ds(t * 1024, 768)],
                    x_hbm.at[pl.ds(xbase + t * 1024, 768)], sem_o))

        for h in h_out:
            h.wait()

    return sc_call, consts_np


def kernel(obs, ego_init, other_init, edge_index):
    batch, obs_dim = obs.shape
    node_num = obs_dim // _INPUT_DIM
    init_dim = ego_init.shape[0]
    num_edges = edge_index.shape[1]
    sc_call, consts_np = _build_sc_call(batch, node_num, _INPUT_DIM,
                                        init_dim, num_edges)
    nrow = node_num * _INPUT_DIM
    pad_r = -nrow % 8
    ncols = batch * node_num
    obsp = jnp.concatenate(
        [obs.T, jnp.zeros((pad_r, batch), obs.dtype)], axis=0)
    obsp = obsp.reshape((nrow + pad_r) // 8, 8, batch // 128, 128) \
               .transpose(0, 2, 1, 3).reshape(-1)
    xphys, ei_flat = sc_call(obsp,
                             jnp.concatenate([ego_init, other_init]),
                             edge_index.reshape(-1),
                             jnp.asarray(consts_np))
    x = xphys.reshape(ncols // 128, 8, 128).transpose(1, 0, 2) \
             .reshape(8, ncols)[:_INPUT_DIM + init_dim].T
    return x, ei_flat.reshape(2, batch * num_edges)
